# Initial kernel scaffold; baseline (speedup 1.0000x reference)
import jax
import jax.numpy as jnp
from jax.experimental import pallas as pl


def kernel(xv, xc, adj_pos, adj_neg, Wv_self, Wc_self, Wv_meta, Wc_meta, qv, qc, Wvc, Wcv, ln_gamma, ln_beta):
    raise NotImplementedError("write your pallas kernel here")


# speedup vs baseline: 1.8912x; 1.8912x over previous
import functools

import jax
import jax.numpy as jnp
from jax import lax
from jax.experimental import pallas as pl
from jax.experimental.pallas import tpu as pltpu
from jax.experimental.pallas import tpu_sc as plsc

_N = 10000
_E = 320000
_D = 128
_NC = 2
_NS = 16
_NW = _NC * _NS
_CH = 128
_NCHUNK = 80
_EPW = _NCHUNK * _CH
_EPAD = _EPW * _NW
_RPT = 632
_AR = _RPT * _NS
_SLICES = ((0, 128), (128, 128), (256, 128), (384, 128), (512, _RPT - 512))
_BN = 400
_NB = _N // _BN


@functools.lru_cache(None)
def _spmm_kernel(d):
    mesh = plsc.VectorSubcoreMesh(core_axis_name="c", subcore_axis_name="s")
    hnc = _NCHUNK // 2
    nit = hnc // 2

    def body(x_hbm, src_hbm, dst_hbm, zro_hbm, out_hbm,
             src_v, dst_v, buf0, buf1, acc, sem0, sem1):
        c = lax.axis_index("c")
        s = lax.axis_index("s")
        w = c * _NS + s
        pltpu.sync_copy(zro_hbm, buf0)
        base = s * _RPT
        for o, r in _SLICES:
            pltpu.sync_copy(buf0.at[pl.ds(0, r)], acc.at[pl.ds(base + o, r)])
        plsc.subcore_barrier()

        for half in range(2):
            pltpu.sync_copy(src_hbm.at[w].at[pl.ds(half * hnc, hnc)], src_v)
            pltpu.sync_copy(dst_hbm.at[w].at[pl.ds(half * hnc, hnc)], dst_v)
            pltpu.async_copy(x_hbm.at[src_v.at[0]], buf0, sem0)

            def step(i, carry):
                j0 = 2 * i
                pltpu.make_async_copy(x_hbm.at[src_v.at[j0]], buf0, sem0).wait()
                pltpu.async_copy(x_hbm.at[src_v.at[j0 + 1]], buf1, sem1)
                pltpu.sync_copy(buf0, acc.at[dst_v.at[j0]], add=True)
                pltpu.make_async_copy(x_hbm.at[src_v.at[j0 + 1]], buf1, sem1).wait()

                @pl.when(i < nit - 1)
                def _():
                    pltpu.async_copy(x_hbm.at[src_v.at[j0 + 2]], buf0, sem0)

                pltpu.sync_copy(buf1, acc.at[dst_v.at[j0 + 1]], add=True)
                return carry

            lax.fori_loop(0, nit, step, 0)
        plsc.subcore_barrier()
        for o, r in _SLICES:
            pltpu.sync_copy(acc.at[pl.ds(base + o, r)], buf0.at[pl.ds(0, r)])
            pltpu.sync_copy(buf0.at[pl.ds(0, r)], out_hbm.at[c].at[pl.ds(base + o, r)])

    return pl.kernel(
        body,
        mesh=mesh,
        out_type=jax.ShapeDtypeStruct((_NC, _AR, d), jnp.float32),
        scratch_types=[
            pltpu.VMEM((_NCHUNK // 2, _CH), jnp.int32),
            pltpu.VMEM((_NCHUNK // 2, _CH), jnp.int32),
            pltpu.VMEM((_CH, d), jnp.float32),
            pltpu.VMEM((_CH, d), jnp.float32),
            pltpu.VMEM_SHARED((_AR, d), jnp.float32),
            pltpu.SemaphoreType.DMA,
            pltpu.SemaphoreType.DMA,
        ],
    )


def _spmm(x, src, dst, zro, d):
    return _spmm_kernel(d)(x, src, dst, zro)


def _norm(part, deg):
    def body(p_ref, d_ref, o_ref):
        dsum = d_ref[0, :, 0] + d_ref[1, :, 0]
        inv = 1.0 / jnp.maximum(dsum, 1.0)
        o_ref[...] = (p_ref[0] + p_ref[1]) * inv[:, None]

    return pl.pallas_call(
        body,
        grid=(_NB,),
        in_specs=[
            pl.BlockSpec((_NC, _BN, _D), lambda i: (0, i, 0)),
            pl.BlockSpec((_NC, _BN, _D), lambda i: (0, i, 0)),
        ],
        out_specs=pl.BlockSpec((_BN, _D), lambda i: (i, 0)),
        out_shape=jax.ShapeDtypeStruct((_N, _D), jnp.float32),
    )(part, deg)


def _meta(h0, h1, h2, h3, degA, degB, W):
    def body(h0r, h1r, h2r, h3r, dar, dbr, wr, hv_ref, cs_ref):
        i = pl.program_id(0)
        invA = 1.0 / jnp.maximum(dar[0, :, 0] + dar[1, :, 0], 1.0)
        invB = 1.0 / jnp.maximum(dbr[0, :, 0] + dbr[1, :, 0], 1.0)

        @pl.when(i == 0)
        def _():
            cs_ref[...] = jnp.zeros((8, _D), jnp.float32)

        hrs = (h0r, h1r, h2r, h3r)
        for p in range(4):
            inv = invA if p < 2 else invB
            hn = (hrs[p][0] + hrs[p][1]) * inv[:, None]
            hv = jnp.tanh(jnp.dot(hn, wr[p], preferred_element_type=jnp.float32))
            hv_ref[p] = hv
            cs_ref[p] = cs_ref[p] + jnp.sum(jnp.tanh(hv), axis=0)

    return pl.pallas_call(
        body,
        grid=(_NB,),
        in_specs=[pl.BlockSpec((_NC, _BN, _D), lambda i: (0, i, 0))] * 4
        + [pl.BlockSpec((_NC, _BN, _D), lambda i: (0, i, 0))] * 2
        + [pl.BlockSpec((4, _D, _D), lambda i: (0, 0, 0))],
        out_specs=[
            pl.BlockSpec((4, _BN, _D), lambda i: (0, i, 0)),
            pl.BlockSpec((8, _D), lambda i: (0, 0)),
        ],
        out_shape=[
            jax.ShapeDtypeStruct((4, _N, _D), jnp.float32),
            jax.ShapeDtypeStruct((8, _D), jnp.float32),
        ],
    )(h0, h1, h2, h3, degA, degB, W)


def _combine(x, Wself, Hv, ua, ub, Wx, cs, q):
    q2 = q.reshape(1, _D)

    def body(x_ref, ws_ref, hv_ref, ua_ref, ub_ref, wx_ref, cs_ref, q_ref, o_ref):
        logits = jnp.sum(cs_ref[...] * q_ref[...], axis=1) * (1.0 / _N)
        mask = lax.broadcasted_iota(jnp.int32, (8,), 0) < 4
        lm = jnp.where(mask, logits, -1e30)
        e = jnp.exp(lm - jnp.max(lm))
        e = jnp.where(mask, e, 0.0)
        b = e / jnp.sum(e)
        acc = jnp.dot(x_ref[...], ws_ref[...], preferred_element_type=jnp.float32)
        acc = acc + jnp.dot(ua_ref[...] + ub_ref[...], wx_ref[...],
                            preferred_element_type=jnp.float32)
        for p in range(4):
            acc = acc + b[p] * hv_ref[p]
        o_ref[...] = jnp.maximum(acc, 0.0)

    return pl.pallas_call(
        body,
        grid=(_NB,),
        in_specs=[
            pl.BlockSpec((_BN, _D), lambda i: (i, 0)),
            pl.BlockSpec((_D, _D), lambda i: (0, 0)),
            pl.BlockSpec((4, _BN, _D), lambda i: (0, i, 0)),
            pl.BlockSpec((_BN, _D), lambda i: (i, 0)),
            pl.BlockSpec((_BN, _D), lambda i: (i, 0)),
            pl.BlockSpec((_D, _D), lambda i: (0, 0)),
            pl.BlockSpec((8, _D), lambda i: (0, 0)),
            pl.BlockSpec((1, _D), lambda i: (0, 0)),
        ],
        out_specs=pl.BlockSpec((_BN, _D), lambda i: (i, 0)),
        out_shape=jax.ShapeDtypeStruct((_N, _D), jnp.float32),
    )(x, Wself, Hv, ua, ub, Wx, cs, q2)


def _ln(x, g, b):
    g2, b2 = g.reshape(1, _D), b.reshape(1, _D)

    def body(x_ref, g_ref, b_ref, o_ref):
        xx = x_ref[...]
        mu = jnp.mean(xx, axis=1, keepdims=True)
        xm = xx - mu
        var = jnp.mean(xm * xm, axis=1, keepdims=True)
        o_ref[...] = g_ref[...] * xm * lax.rsqrt(var + 1e-5) + b_ref[...]

    return pl.pallas_call(
        body,
        grid=(_NB,),
        in_specs=[
            pl.BlockSpec((_BN, _D), lambda i: (i, 0)),
            pl.BlockSpec((1, _D), lambda i: (0, 0)),
            pl.BlockSpec((1, _D), lambda i: (0, 0)),
        ],
        out_specs=pl.BlockSpec((_BN, _D), lambda i: (i, 0)),
        out_shape=jax.ShapeDtypeStruct((_N, _D), jnp.float32),
    )(x, g2, b2)


def _prep_edges(src, dst):
    pad = _EPAD - _E
    s = jnp.concatenate([src, jnp.zeros((pad,), jnp.int32)])
    t = jnp.concatenate([dst, jnp.full((pad,), _N, jnp.int32)])
    return s.reshape(_NW, _NCHUNK, _CH), t.reshape(_NW, _NCHUNK, _CH)


def kernel(xv, xc, adj_pos, adj_neg, Wv_self, Wc_self, Wv_meta, Wc_meta,
           qv, qc, Wvc, Wcv, ln_gamma, ln_beta):
    z128 = jnp.zeros((_CH, _D), jnp.float32)
    ones = jnp.ones((_N, _D), jnp.float32)

    P_s, P_d = _prep_edges(adj_pos[1], adj_pos[0])
    PT_s, PT_d = _prep_edges(adj_pos[0], adj_pos[1])
    G_s, G_d = _prep_edges(adj_neg[1], adj_neg[0])
    GT_s, GT_d = _prep_edges(adj_neg[0], adj_neg[1])

    deg_p0 = _spmm(ones, P_s, P_d, z128, _D)
    deg_p1 = _spmm(ones, PT_s, PT_d, z128, _D)
    deg_n0 = _spmm(ones, G_s, G_d, z128, _D)
    deg_n1 = _spmm(ones, GT_s, GT_d, z128, _D)

    L = Wv_self.shape[0]
    for l in range(L):
        t_pos = _spmm(xv, P_s, P_d, z128, _D)
        t_neg = _spmm(xv, G_s, G_d, z128, _D)
        u_pos = _spmm(xc, PT_s, PT_d, z128, _D)
        u_neg = _spmm(xc, GT_s, GT_d, z128, _D)
        t_pos_n = _norm(t_pos, deg_p0)
        t_neg_n = _norm(t_neg, deg_n0)
        u_pos_n = _norm(u_pos, deg_p1)
        u_neg_n = _norm(u_neg, deg_n1)
        hv0 = _spmm(t_pos_n, PT_s, PT_d, z128, _D)
        hv1 = _spmm(t_neg_n, PT_s, PT_d, z128, _D)
        hv2 = _spmm(t_pos_n, GT_s, GT_d, z128, _D)
        hv3 = _spmm(t_neg_n, GT_s, GT_d, z128, _D)
        hc0 = _spmm(u_pos_n, P_s, P_d, z128, _D)
        hc1 = _spmm(u_neg_n, P_s, P_d, z128, _D)
        hc2 = _spmm(u_pos_n, G_s, G_d, z128, _D)
        hc3 = _spmm(u_neg_n, G_s, G_d, z128, _D)
        Hv, csv = _meta(hv0, hv1, hv2, hv3, deg_p1, deg_n1, Wv_meta[l])
        Hc, csc = _meta(hc0, hc1, hc2, hc3, deg_p0, deg_n0, Wc_meta[l])
        xv = _combine(xv, Wv_self[l], Hv, u_pos_n, u_neg_n, Wvc[l], csv, qv[l])
        xc = _combine(xc, Wc_self[l], Hc, t_pos_n, t_neg_n, Wcv[l], csc, qc[l])
    return _ln(xv, ln_gamma, ln_beta), _ln(xc, ln_gamma, ln_beta)


# speedup vs baseline: 2.1559x; 1.1400x over previous
import functools

import jax
import jax.numpy as jnp
from jax import lax
from jax.experimental import pallas as pl
from jax.experimental.pallas import tpu as pltpu
from jax.experimental.pallas import tpu_sc as plsc

_N = 10000
_E = 320000
_D = 128
_NC = 2
_NS = 16
_NW = _NC * _NS
_CH = 128
_NCHUNK = 80
_EPW = _NCHUNK * _CH
_EPAD = _EPW * _NW
_RPT = 632
_AR = _RPT * _NS
_SLICES = ((0, 128), (128, 128), (256, 128), (384, 128), (512, _RPT - 512))
_BN = 400
_NB = _N // _BN

_P, _PT, _G, _GT = 0, 1, 2, 3


@functools.lru_cache(None)
def _sc_multi(plan, nx):
    mesh = plsc.VectorSubcoreMesh(core_axis_name="c", subcore_axis_name="s")
    hnc = _NCHUNK // 2
    nit = hnc // 2
    nout = len(plan)

    def body(x_hbm, src_hbm, dst_hbm, cst_hbm, out_hbm,
             src_v, dst_v, buf0, buf1, acc, sem0, sem1):
        c = lax.axis_index("c")
        s = lax.axis_index("s")
        w = c * _NS + s
        base = s * _RPT

        def zero_slice():
            pltpu.sync_copy(cst_hbm.at[0], buf0)
            for o, r in _SLICES:
                pltpu.sync_copy(buf0.at[pl.ds(0, r)], acc.at[pl.ds(base + o, r)])

        zero_slice()
        plsc.subcore_barrier()

        for oi, (xi, si) in enumerate(plan):
            if xi is None:
                pltpu.sync_copy(cst_hbm.at[1], buf1)
                for half in range(2):
                    pltpu.sync_copy(dst_hbm.at[si].at[w].at[pl.ds(half * hnc, hnc)],
                                    dst_v)

                    def dstep(i, carry):
                        pltpu.sync_copy(buf1, acc.at[dst_v.at[i]], add=True)
                        return carry

                    lax.fori_loop(0, hnc, dstep, 0)
            else:
                xt = x_hbm.at[xi]
                for half in range(2):
                    pltpu.sync_copy(src_hbm.at[si].at[w].at[pl.ds(half * hnc, hnc)],
                                    src_v)
                    pltpu.sync_copy(dst_hbm.at[si].at[w].at[pl.ds(half * hnc, hnc)],
                                    dst_v)
                    pltpu.async_copy(xt.at[src_v.at[0]], buf0, sem0)

                    def step(i, carry):
                        j0 = 2 * i
                        pltpu.make_async_copy(xt.at[src_v.at[j0]], buf0, sem0).wait()
                        pltpu.async_copy(xt.at[src_v.at[j0 + 1]], buf1, sem1)
                        pltpu.sync_copy(buf0, acc.at[dst_v.at[j0]], add=True)
                        pltpu.make_async_copy(xt.at[src_v.at[j0 + 1]], buf1,
                                              sem1).wait()

                        @pl.when(i < nit - 1)
                        def _():
                            pltpu.async_copy(xt.at[src_v.at[j0 + 2]], buf0, sem0)

                        pltpu.sync_copy(buf1, acc.at[dst_v.at[j0 + 1]], add=True)
                        return carry

                    lax.fori_loop(0, nit, step, 0)
            plsc.subcore_barrier()
            ob = out_hbm.at[oi].at[c]
            for o, r in _SLICES:
                pltpu.sync_copy(acc.at[pl.ds(base + o, r)], buf0.at[pl.ds(0, r)])
                pltpu.sync_copy(buf0.at[pl.ds(0, r)], ob.at[pl.ds(base + o, r)])
            if oi < nout - 1:
                zero_slice()
                plsc.subcore_barrier()

    return pl.kernel(
        body,
        mesh=mesh,
        out_type=jax.ShapeDtypeStruct((nout, _NC, _AR, _D), jnp.float32),
        scratch_types=[
            pltpu.VMEM((_NCHUNK // 2, _CH), jnp.int32),
            pltpu.VMEM((_NCHUNK // 2, _CH), jnp.int32),
            pltpu.VMEM((_CH, _D), jnp.float32),
            pltpu.VMEM((_CH, _D), jnp.float32),
            pltpu.VMEM_SHARED((_AR, _D), jnp.float32),
            pltpu.SemaphoreType.DMA,
            pltpu.SemaphoreType.DMA,
        ],
    )


def _norm4(parts, degs, deg_base):
    def body(p_ref, d_ref, o_ref):
        dsum = d_ref[0, 0, :, 0] + d_ref[0, 1, :, 0]
        inv = 1.0 / jnp.maximum(dsum, 1.0)
        o_ref[0] = (p_ref[0, 0] + p_ref[0, 1]) * inv[:, None]

    return pl.pallas_call(
        body,
        grid=(4, _NB),
        in_specs=[
            pl.BlockSpec((1, _NC, _BN, _D), lambda p, i: (p, 0, i, 0)),
            pl.BlockSpec((1, _NC, _BN, _D), lambda p, i: (deg_base + p, 0, i, 0)),
        ],
        out_specs=pl.BlockSpec((1, _BN, _D), lambda p, i: (p, i, 0)),
        out_shape=jax.ShapeDtypeStruct((4, _N, _D), jnp.float32),
    )(parts, degs)


def _meta(big2, hbase, degs, dA, dB, W):
    def body(h0r, h1r, h2r, h3r, dar, dbr, wr, hv_ref, cs_ref):
        i = pl.program_id(0)
        invA = 1.0 / jnp.maximum(dar[0, 0, :, 0] + dar[0, 1, :, 0], 1.0)
        invB = 1.0 / jnp.maximum(dbr[0, 0, :, 0] + dbr[0, 1, :, 0], 1.0)

        @pl.when(i == 0)
        def _():
            cs_ref[...] = jnp.zeros((8, _D), jnp.float32)

        hrs = (h0r, h1r, h2r, h3r)
        for p in range(4):
            inv = invA if p < 2 else invB
            hn = (hrs[p][0, 0] + hrs[p][0, 1]) * inv[:, None]
            hv = jnp.tanh(jnp.dot(hn, wr[p], preferred_element_type=jnp.float32))
            hv_ref[p] = hv
            cs_ref[p] = cs_ref[p] + jnp.sum(jnp.tanh(hv), axis=0)

    return pl.pallas_call(
        body,
        grid=(_NB,),
        in_specs=[pl.BlockSpec((1, _NC, _BN, _D),
                               lambda i, p=p: (hbase + p, 0, i, 0))
                  for p in range(4)]
        + [pl.BlockSpec((1, _NC, _BN, _D), lambda i: (dA, 0, i, 0)),
           pl.BlockSpec((1, _NC, _BN, _D), lambda i: (dB, 0, i, 0)),
           pl.BlockSpec((4, _D, _D), lambda i: (0, 0, 0))],
        out_specs=[
            pl.BlockSpec((4, _BN, _D), lambda i: (0, i, 0)),
            pl.BlockSpec((8, _D), lambda i: (0, 0)),
        ],
        out_shape=[
            jax.ShapeDtypeStruct((4, _N, _D), jnp.float32),
            jax.ShapeDtypeStruct((8, _D), jnp.float32),
        ],
    )(big2, big2, big2, big2, degs, degs, W)


def _combine(x, Wself, Hv, xn, ua_i, ub_i, Wx, cs, q):
    q2 = q.reshape(1, _D)

    def body(x_ref, ws_ref, hv_ref, ua_ref, ub_ref, wx_ref, cs_ref, q_ref, o_ref):
        logits = jnp.sum(cs_ref[...] * q_ref[...], axis=1) * (1.0 / _N)
        mask = lax.broadcasted_iota(jnp.int32, (8,), 0) < 4
        lm = jnp.where(mask, logits, -1e30)
        e = jnp.exp(lm - jnp.max(lm))
        e = jnp.where(mask, e, 0.0)
        b = e / jnp.sum(e)
        acc = jnp.dot(x_ref[...], ws_ref[...], preferred_element_type=jnp.float32)
        acc = acc + jnp.dot(ua_ref[0] + ub_ref[0], wx_ref[...],
                            preferred_element_type=jnp.float32)
        for p in range(4):
            acc = acc + b[p] * hv_ref[p]
        o_ref[...] = jnp.maximum(acc, 0.0)

    return pl.pallas_call(
        body,
        grid=(_NB,),
        in_specs=[
            pl.BlockSpec((_BN, _D), lambda i: (i, 0)),
            pl.BlockSpec((_D, _D), lambda i: (0, 0)),
            pl.BlockSpec((4, _BN, _D), lambda i: (0, i, 0)),
            pl.BlockSpec((1, _BN, _D), lambda i: (ua_i, i, 0)),
            pl.BlockSpec((1, _BN, _D), lambda i: (ub_i, i, 0)),
            pl.BlockSpec((_D, _D), lambda i: (0, 0)),
            pl.BlockSpec((8, _D), lambda i: (0, 0)),
            pl.BlockSpec((1, _D), lambda i: (0, 0)),
        ],
        out_specs=pl.BlockSpec((_BN, _D), lambda i: (i, 0)),
        out_shape=jax.ShapeDtypeStruct((_N, _D), jnp.float32),
    )(x, Wself, Hv, xn, xn, Wx, cs, q2)


def _ln(x, g, b):
    g2, b2 = g.reshape(1, _D), b.reshape(1, _D)

    def body(x_ref, g_ref, b_ref, o_ref):
        xx = x_ref[...]
        mu = jnp.mean(xx, axis=1, keepdims=True)
        xm = xx - mu
        var = jnp.mean(xm * xm, axis=1, keepdims=True)
        o_ref[...] = g_ref[...] * xm * lax.rsqrt(var + 1e-5) + b_ref[...]

    return pl.pallas_call(
        body,
        grid=(_NB,),
        in_specs=[
            pl.BlockSpec((_BN, _D), lambda i: (i, 0)),
            pl.BlockSpec((1, _D), lambda i: (0, 0)),
            pl.BlockSpec((1, _D), lambda i: (0, 0)),
        ],
        out_specs=pl.BlockSpec((_BN, _D), lambda i: (i, 0)),
        out_shape=jax.ShapeDtypeStruct((_N, _D), jnp.float32),
    )(x, g2, b2)


def _prep_edges(src, dst):
    pad = _EPAD - _E
    s = jnp.concatenate([src, jnp.zeros((pad,), jnp.int32)])
    t = jnp.concatenate([dst, jnp.full((pad,), _N, jnp.int32)])
    return s.reshape(_NW, _NCHUNK, _CH), t.reshape(_NW, _NCHUNK, _CH)


_PLAN1_L0 = ((0, _P), (0, _G), (1, _PT), (1, _GT),
             (None, _P), (None, _G), (None, _PT), (None, _GT))
_PLAN1 = _PLAN1_L0[:4]
_PLAN2 = ((0, _PT), (1, _PT), (0, _GT), (1, _GT),
          (2, _P), (3, _P), (2, _G), (3, _G))


def kernel(xv, xc, adj_pos, adj_neg, Wv_self, Wc_self, Wv_meta, Wc_meta,
           qv, qc, Wvc, Wcv, ln_gamma, ln_beta):
    cst = jnp.stack([jnp.zeros((_CH, _D), jnp.float32),
                     jnp.ones((_CH, _D), jnp.float32)])

    P_s, P_d = _prep_edges(adj_pos[1], adj_pos[0])
    PT_s, PT_d = _prep_edges(adj_pos[0], adj_pos[1])
    G_s, G_d = _prep_edges(adj_neg[1], adj_neg[0])
    GT_s, GT_d = _prep_edges(adj_neg[0], adj_neg[1])
    src_all = jnp.stack([P_s, PT_s, G_s, GT_s])
    dst_all = jnp.stack([P_d, PT_d, G_d, GT_d])

    L = Wv_self.shape[0]
    degs = None
    for l in range(L):
        tabs = jnp.stack([xv, xc])
        plan1 = _PLAN1_L0 if l == 0 else _PLAN1
        big1 = _sc_multi(plan1, 2)(tabs, src_all, dst_all, cst)
        if l == 0:
            degs = big1
        xn = _norm4(big1, degs, 4)
        big2 = _sc_multi(_PLAN2, 4)(xn, src_all, dst_all, cst)
        Hv, csv = _meta(big2, 0, degs, 6, 7, Wv_meta[l])
        Hc, csc = _meta(big2, 4, degs, 4, 5, Wc_meta[l])
        xv = _combine(xv, Wv_self[l], Hv, xn, 2, 3, Wvc[l], csv, qv[l])
        xc = _combine(xc, Wc_self[l], Hc, xn, 0, 1, Wcv[l], csc, qc[l])
    return _ln(xv, ln_gamma, ln_beta), _ln(xc, ln_gamma, ln_beta)


# speedup vs baseline: 2.2343x; 1.0364x over previous
import functools

import jax
import jax.numpy as jnp
from jax import lax
from jax.experimental import pallas as pl
from jax.experimental.pallas import tpu as pltpu
from jax.experimental.pallas import tpu_sc as plsc

_N = 10000
_E = 320000
_D = 128
_NC = 2
_NS = 16
_NW = _NC * _NS
_CH = 64
_NCHUNK = 160
_EPW = _NCHUNK * _CH
_EPAD = _EPW * _NW
_RPT = 632
_AR = _RPT * _NS
_SLICES = tuple((o, min(64, _RPT - o)) for o in range(0, _RPT, 64))
_BN = 400
_NB = _N // _BN

_P, _PT, _G, _GT = 0, 1, 2, 3


@functools.lru_cache(None)
def _sc_multi(plan, nx):
    mesh = plsc.VectorSubcoreMesh(core_axis_name="c", subcore_axis_name="s")
    qnc = _NCHUNK // 4
    nout = len(plan)

    def body(x_hbm, src_hbm, dst_hbm, cst_hbm, out_hbm,
             src_v, dst_v, buf0, buf1, buf2, buf3, acc,
             sem0, sem1, sem2, sem3):
        bufs = (buf0, buf1, buf2, buf3)
        sems = (sem0, sem1, sem2, sem3)
        c = lax.axis_index("c")
        s = lax.axis_index("s")
        w = c * _NS + s
        base = s * _RPT

        def zero_slice():
            pltpu.sync_copy(cst_hbm.at[0], buf0)
            for o, r in _SLICES:
                pltpu.sync_copy(buf0.at[pl.ds(0, r)], acc.at[pl.ds(base + o, r)])

        zero_slice()
        plsc.subcore_barrier()

        for oi, (xi, si) in enumerate(plan):
            if xi is None:
                pltpu.sync_copy(cst_hbm.at[1], buf1)
                for q in range(4):
                    pltpu.sync_copy(dst_hbm.at[si].at[w].at[pl.ds(q * qnc, qnc)],
                                    dst_v)

                    def dstep(i, carry):
                        pltpu.sync_copy(buf1, acc.at[dst_v.at[i]], add=True)
                        return carry

                    lax.fori_loop(0, qnc, dstep, 0)
            else:
                xt = x_hbm.at[xi]
                for q in range(4):
                    pltpu.sync_copy(src_hbm.at[si].at[w].at[pl.ds(q * qnc, qnc)],
                                    src_v)
                    pltpu.sync_copy(dst_hbm.at[si].at[w].at[pl.ds(q * qnc, qnc)],
                                    dst_v)
                    for k in range(4):
                        pltpu.async_copy(xt.at[src_v.at[k]], bufs[k], sems[k])

                    def step(i, carry):
                        for k in range(4):
                            j = 4 * i + k
                            pltpu.make_async_copy(xt.at[src_v.at[j]], bufs[k],
                                                  sems[k]).wait()
                            pltpu.sync_copy(bufs[k], acc.at[dst_v.at[j]],
                                            add=True)

                            @pl.when(j + 4 < qnc)
                            def _():
                                pltpu.async_copy(xt.at[src_v.at[j + 4]],
                                                 bufs[k], sems[k])

                        return carry

                    lax.fori_loop(0, qnc // 4, step, 0)
            plsc.subcore_barrier()
            ob = out_hbm.at[oi].at[c]
            for o, r in _SLICES:
                pltpu.sync_copy(acc.at[pl.ds(base + o, r)], buf0.at[pl.ds(0, r)])
                pltpu.sync_copy(buf0.at[pl.ds(0, r)], ob.at[pl.ds(base + o, r)])
            if oi < nout - 1:
                zero_slice()
                plsc.subcore_barrier()

    return pl.kernel(
        body,
        mesh=mesh,
        out_type=jax.ShapeDtypeStruct((nout, _NC, _AR, _D), jnp.float32),
        scratch_types=[
            pltpu.VMEM((_NCHUNK // 4, _CH), jnp.int32),
            pltpu.VMEM((_NCHUNK // 4, _CH), jnp.int32),
            pltpu.VMEM((_CH, _D), jnp.float32),
            pltpu.VMEM((_CH, _D), jnp.float32),
            pltpu.VMEM((_CH, _D), jnp.float32),
            pltpu.VMEM((_CH, _D), jnp.float32),
            pltpu.VMEM_SHARED((_AR, _D), jnp.float32),
            pltpu.SemaphoreType.DMA,
            pltpu.SemaphoreType.DMA,
            pltpu.SemaphoreType.DMA,
            pltpu.SemaphoreType.DMA,
        ],
    )


def _norm4(parts, degs, deg_base):
    def body(p_ref, d_ref, o_ref):
        dsum = d_ref[0, 0, :, 0] + d_ref[0, 1, :, 0]
        inv = 1.0 / jnp.maximum(dsum, 1.0)
        o_ref[0] = (p_ref[0, 0] + p_ref[0, 1]) * inv[:, None]

    return pl.pallas_call(
        body,
        grid=(4, _NB),
        in_specs=[
            pl.BlockSpec((1, _NC, _BN, _D), lambda p, i: (p, 0, i, 0)),
            pl.BlockSpec((1, _NC, _BN, _D), lambda p, i: (deg_base + p, 0, i, 0)),
        ],
        out_specs=pl.BlockSpec((1, _BN, _D), lambda p, i: (p, i, 0)),
        out_shape=jax.ShapeDtypeStruct((4, _N, _D), jnp.float32),
    )(parts, degs)


def _meta(big2, hbase, degs, dA, dB, W):
    def body(h0r, h1r, h2r, h3r, dar, dbr, wr, hv_ref, cs_ref):
        i = pl.program_id(0)
        invA = 1.0 / jnp.maximum(dar[0, 0, :, 0] + dar[0, 1, :, 0], 1.0)
        invB = 1.0 / jnp.maximum(dbr[0, 0, :, 0] + dbr[0, 1, :, 0], 1.0)

        @pl.when(i == 0)
        def _():
            cs_ref[...] = jnp.zeros((8, _D), jnp.float32)

        hrs = (h0r, h1r, h2r, h3r)
        for p in range(4):
            inv = invA if p < 2 else invB
            hn = (hrs[p][0, 0] + hrs[p][0, 1]) * inv[:, None]
            hv = jnp.tanh(jnp.dot(hn, wr[p], preferred_element_type=jnp.float32))
            hv_ref[p] = hv
            cs_ref[p] = cs_ref[p] + jnp.sum(jnp.tanh(hv), axis=0)

    return pl.pallas_call(
        body,
        grid=(_NB,),
        in_specs=[pl.BlockSpec((1, _NC, _BN, _D),
                               lambda i, p=p: (hbase + p, 0, i, 0))
                  for p in range(4)]
        + [pl.BlockSpec((1, _NC, _BN, _D), lambda i: (dA, 0, i, 0)),
           pl.BlockSpec((1, _NC, _BN, _D), lambda i: (dB, 0, i, 0)),
           pl.BlockSpec((4, _D, _D), lambda i: (0, 0, 0))],
        out_specs=[
            pl.BlockSpec((4, _BN, _D), lambda i: (0, i, 0)),
            pl.BlockSpec((8, _D), lambda i: (0, 0)),
        ],
        out_shape=[
            jax.ShapeDtypeStruct((4, _N, _D), jnp.float32),
            jax.ShapeDtypeStruct((8, _D), jnp.float32),
        ],
    )(big2, big2, big2, big2, degs, degs, W)


def _combine(x, Wself, Hv, xn, ua_i, ub_i, Wx, cs, q):
    q2 = q.reshape(1, _D)

    def body(x_ref, ws_ref, hv_ref, ua_ref, ub_ref, wx_ref, cs_ref, q_ref, o_ref):
        logits = jnp.sum(cs_ref[...] * q_ref[...], axis=1) * (1.0 / _N)
        mask = lax.broadcasted_iota(jnp.int32, (8,), 0) < 4
        lm = jnp.where(mask, logits, -1e30)
        e = jnp.exp(lm - jnp.max(lm))
        e = jnp.where(mask, e, 0.0)
        b = e / jnp.sum(e)
        acc = jnp.dot(x_ref[...], ws_ref[...], preferred_element_type=jnp.float32)
        acc = acc + jnp.dot(ua_ref[0] + ub_ref[0], wx_ref[...],
                            preferred_element_type=jnp.float32)
        for p in range(4):
            acc = acc + b[p] * hv_ref[p]
        o_ref[...] = jnp.maximum(acc, 0.0)

    return pl.pallas_call(
        body,
        grid=(_NB,),
        in_specs=[
            pl.BlockSpec((_BN, _D), lambda i: (i, 0)),
            pl.BlockSpec((_D, _D), lambda i: (0, 0)),
            pl.BlockSpec((4, _BN, _D), lambda i: (0, i, 0)),
            pl.BlockSpec((1, _BN, _D), lambda i: (ua_i, i, 0)),
            pl.BlockSpec((1, _BN, _D), lambda i: (ub_i, i, 0)),
            pl.BlockSpec((_D, _D), lambda i: (0, 0)),
            pl.BlockSpec((8, _D), lambda i: (0, 0)),
            pl.BlockSpec((1, _D), lambda i: (0, 0)),
        ],
        out_specs=pl.BlockSpec((_BN, _D), lambda i: (i, 0)),
        out_shape=jax.ShapeDtypeStruct((_N, _D), jnp.float32),
    )(x, Wself, Hv, xn, xn, Wx, cs, q2)


def _ln(x, g, b):
    g2, b2 = g.reshape(1, _D), b.reshape(1, _D)

    def body(x_ref, g_ref, b_ref, o_ref):
        xx = x_ref[...]
        mu = jnp.mean(xx, axis=1, keepdims=True)
        xm = xx - mu
        var = jnp.mean(xm * xm, axis=1, keepdims=True)
        o_ref[...] = g_ref[...] * xm * lax.rsqrt(var + 1e-5) + b_ref[...]

    return pl.pallas_call(
        body,
        grid=(_NB,),
        in_specs=[
            pl.BlockSpec((_BN, _D), lambda i: (i, 0)),
            pl.BlockSpec((1, _D), lambda i: (0, 0)),
            pl.BlockSpec((1, _D), lambda i: (0, 0)),
        ],
        out_specs=pl.BlockSpec((_BN, _D), lambda i: (i, 0)),
        out_shape=jax.ShapeDtypeStruct((_N, _D), jnp.float32),
    )(x, g2, b2)


def _prep_edges(src, dst):
    pad = _EPAD - _E
    s = jnp.concatenate([src, jnp.zeros((pad,), jnp.int32)])
    t = jnp.concatenate([dst, jnp.full((pad,), _N, jnp.int32)])
    return s.reshape(_NW, _NCHUNK, _CH), t.reshape(_NW, _NCHUNK, _CH)


_PLAN1_L0 = ((0, _P), (0, _G), (1, _PT), (1, _GT),
             (None, _P), (None, _G), (None, _PT), (None, _GT))
_PLAN1 = _PLAN1_L0[:4]
_PLAN2 = ((0, _PT), (1, _PT), (0, _GT), (1, _GT),
          (2, _P), (3, _P), (2, _G), (3, _G))


def kernel(xv, xc, adj_pos, adj_neg, Wv_self, Wc_self, Wv_meta, Wc_meta,
           qv, qc, Wvc, Wcv, ln_gamma, ln_beta):
    cst = jnp.stack([jnp.zeros((_CH, _D), jnp.float32),
                     jnp.ones((_CH, _D), jnp.float32)])

    P_s, P_d = _prep_edges(adj_pos[1], adj_pos[0])
    PT_s, PT_d = _prep_edges(adj_pos[0], adj_pos[1])
    G_s, G_d = _prep_edges(adj_neg[1], adj_neg[0])
    GT_s, GT_d = _prep_edges(adj_neg[0], adj_neg[1])
    src_all = jnp.stack([P_s, PT_s, G_s, GT_s])
    dst_all = jnp.stack([P_d, PT_d, G_d, GT_d])

    L = Wv_self.shape[0]
    degs = None
    for l in range(L):
        tabs = jnp.stack([xv, xc])
        plan1 = _PLAN1_L0 if l == 0 else _PLAN1
        big1 = _sc_multi(plan1, 2)(tabs, src_all, dst_all, cst)
        if l == 0:
            degs = big1
        xn = _norm4(big1, degs, 4)
        big2 = _sc_multi(_PLAN2, 4)(xn, src_all, dst_all, cst)
        Hv, csv = _meta(big2, 0, degs, 6, 7, Wv_meta[l])
        Hc, csc = _meta(big2, 4, degs, 4, 5, Wc_meta[l])
        xv = _combine(xv, Wv_self[l], Hv, xn, 2, 3, Wvc[l], csv, qv[l])
        xc = _combine(xc, Wc_self[l], Hc, xn, 0, 1, Wcv[l], csc, qc[l])
    return _ln(xv, ln_gamma, ln_beta), _ln(xc, ln_gamma, ln_beta)
